# baseline (speedup 1.0000x reference)
import functools

import jax
import jax.numpy as jnp
from jax import lax
from jax.experimental import pallas as pl
from jax.experimental.pallas import tpu as pltpu
from jax.experimental.pallas import tpu_sc as plsc

NC, NS = 2, 16
NW = NC * NS
B = 4096 * 200
D = 64
BPW = B // NW
SUB = 128
K = 8
CHUNK = SUB * K
NCHUNK = BPW // CHUNK

_mesh = plsc.VectorSubcoreMesh(core_axis_name="c", subcore_axis_name="s")


@functools.partial(
    pl.kernel,
    out_type=jax.ShapeDtypeStruct((B, D), jnp.float32),
    mesh=_mesh,
    compiler_params=pltpu.CompilerParams(use_tc_tiling_on_sc=False),
    scratch_types=[
        pltpu.VMEM((CHUNK,), jnp.int32),
        pltpu.VMEM((CHUNK, D), jnp.float32),
        pltpu.SemaphoreType.DMA,
        pltpu.SemaphoreType.DMA,
    ],
)
def _gather_kernel(idx_hbm, w_hbm, out_hbm, idx_v, rows_v, sem_i, sem_g):
    wid = lax.axis_index("s") * NC + lax.axis_index("c")
    base = wid * BPW

    @pl.loop(0, NCHUNK)
    def _chunk(g):
        off = base + g * CHUNK
        pltpu.async_copy(idx_hbm.at[pl.ds(off, CHUNK)], idx_v, sem_i).wait()
        copies = []
        for j in range(K):
            copies.append(pltpu.async_copy(
                w_hbm.at[idx_v.at[pl.ds(j * SUB, SUB)]],
                rows_v.at[pl.ds(j * SUB, SUB)], sem_g,
            ))
        for c in copies:
            c.wait()
        pltpu.sync_copy(rows_v, out_hbm.at[pl.ds(off, CHUNK)])


def kernel(idx, weight):
    flat = idx.reshape(-1).astype(jnp.int32)
    out = _gather_kernel(flat, weight)
    return out.reshape(idx.shape + (weight.shape[-1],))


# speedup vs baseline: 1.0154x; 1.0154x over previous
import functools

import jax
import jax.numpy as jnp
from jax import lax
from jax.experimental import pallas as pl
from jax.experimental.pallas import tpu as pltpu
from jax.experimental.pallas import tpu_sc as plsc

NC, NS = 2, 16
NW = NC * NS
B = 4096 * 200
D = 64
BPW = B // NW
SUB = 128
K = 4
CHUNK = SUB * K
NCHUNK = BPW // CHUNK
NBUF = 3

_mesh = plsc.VectorSubcoreMesh(core_axis_name="c", subcore_axis_name="s")


@functools.partial(
    pl.kernel,
    out_type=jax.ShapeDtypeStruct((B, D), jnp.float32),
    mesh=_mesh,
    compiler_params=pltpu.CompilerParams(use_tc_tiling_on_sc=False),
    scratch_types=(
        [pltpu.VMEM((CHUNK,), jnp.int32) for _ in range(NBUF)]
        + [pltpu.VMEM((CHUNK, D), jnp.float32) for _ in range(NBUF)]
        + [pltpu.SemaphoreType.DMA for _ in range(3 * NBUF)]
    ),
)
def _gather_kernel(idx_hbm, w_hbm, out_hbm, *scratch):
    idx_v = scratch[:NBUF]
    rows_v = scratch[NBUF:2 * NBUF]
    sem_i = scratch[2 * NBUF:3 * NBUF]
    sem_g = scratch[3 * NBUF:4 * NBUF]
    sem_s = scratch[4 * NBUF:5 * NBUF]

    wid = lax.axis_index("s") * NC + lax.axis_index("c")
    base = wid * BPW

    def wait_idx(b):
        pltpu.make_async_copy(
            idx_hbm.at[pl.ds(0, CHUNK)], idx_v[b], sem_i[b]).wait()

    def wait_gathers(b):
        pltpu.make_async_copy(
            w_hbm.at[pl.ds(0, CHUNK)], rows_v[b], sem_g[b]).wait()

    def wait_store(b):
        pltpu.make_async_copy(
            rows_v[b], out_hbm.at[pl.ds(0, CHUNK)], sem_s[b]).wait()

    def start_idx(b, t):
        pltpu.async_copy(
            idx_hbm.at[pl.ds(base + t * CHUNK, CHUNK)], idx_v[b], sem_i[b])

    def start_gathers(b):
        for j in range(K):
            pltpu.async_copy(
                w_hbm.at[idx_v[b].at[pl.ds(j * SUB, SUB)]],
                rows_v[b].at[pl.ds(j * SUB, SUB)], sem_g[b])

    def start_store(b, t):
        pltpu.async_copy(
            rows_v[b], out_hbm.at[pl.ds(base + t * CHUNK, CHUNK)], sem_s[b])

    for b in range(NBUF):
        start_idx(b, b)
    for b in range(NBUF):
        wait_idx(b)
        start_gathers(b)

    @pl.loop(0, NCHUNK)
    def _chunk(t):
        b = lax.rem(t, NBUF)

        def per_buf(bb):
            @pl.when(b == bb)
            def _():
                wait_gathers(bb)
                start_store(bb, t)

                @pl.when(t + NBUF < NCHUNK)
                def _():
                    start_idx(bb, t + NBUF)
                    wait_store(bb)
                    wait_idx(bb)
                    start_gathers(bb)

        for bb in range(NBUF):
            per_buf(bb)

    for b in range(NBUF):
        wait_store(b)


def kernel(idx, weight):
    flat = idx.reshape(-1).astype(jnp.int32)
    out = _gather_kernel(flat, weight)
    return out.reshape(idx.shape + (weight.shape[-1],))


# speedup vs baseline: 2.4719x; 2.4344x over previous
import functools

import jax
import jax.numpy as jnp
from jax import lax
from jax.experimental import pallas as pl
from jax.experimental.pallas import tpu as pltpu
from jax.experimental.pallas import tpu_sc as plsc

VOCAB = 1000000
TBLK = 17920
NTB = 28
OFF = TBLK * NTB
VOCAB2 = 2 * OFF
NC, NS = 2, 16
NW = NC * NS
B = 4096 * 200
D = 64
BPW = B // NW
SUB = 128
K = 4
CHUNK = SUB * K
NCHUNK = BPW // CHUNK
NBUF = 3
L = 16

_mesh = plsc.VectorSubcoreMesh(core_axis_name="c", subcore_axis_name="s")


def _relayout_body(lo_ref, hi_ref, out_ref):
    out_ref[:, 0:D] = jnp.swapaxes(lo_ref[...], 0, 1)
    out_ref[:, D:2 * D] = jnp.swapaxes(hi_ref[...], 0, 1)


def _relayout(wt):
    return pl.pallas_call(
        _relayout_body,
        grid=(NTB,),
        in_specs=[
            pl.BlockSpec((D, TBLK), lambda i: (0, i)),
            pl.BlockSpec((D, TBLK), lambda i: (0, i + NTB)),
        ],
        out_specs=pl.BlockSpec((TBLK, 2 * D), lambda i: (i, 0)),
        out_shape=jax.ShapeDtypeStruct((OFF, 2 * D), jnp.float32),
    )(wt, wt)


def _make_gather(half, nrows):
  bpw = nrows // NW
  nchunk = bpw // CHUNK

  @functools.partial(
      pl.kernel,
      out_type=jax.ShapeDtypeStruct((nrows, D), jnp.float32),
      mesh=_mesh,
      compiler_params=pltpu.CompilerParams(use_tc_tiling_on_sc=False),
      scratch_types=(
          [pltpu.VMEM((CHUNK,), jnp.int32) for _ in range(NBUF)]
          + [pltpu.VMEM((CHUNK, D), jnp.float32) for _ in range(NBUF)]
          + [pltpu.SemaphoreType.DMA for _ in range(3 * NBUF)]
      ),
  )
  def _gather_kernel(idx_hbm, w_hbm, out_hbm, *scratch):
    idx_v = scratch[:NBUF]
    rows_v = scratch[NBUF:2 * NBUF]
    sem_i = scratch[2 * NBUF:3 * NBUF]
    sem_g = scratch[3 * NBUF:4 * NBUF]
    sem_s = scratch[4 * NBUF:5 * NBUF]

    wid = lax.axis_index("s") * NC + lax.axis_index("c")
    base = wid * bpw
    ibase = half * nrows + base

    def wait_idx(b):
        pltpu.make_async_copy(
            idx_hbm.at[pl.ds(0, CHUNK)], idx_v[b], sem_i[b]).wait()

    def remap_idx(b):
        for i in range(CHUNK // L):
            v = idx_v[b][pl.ds(i * L, L)]
            two = v + v
            idx_v[b][pl.ds(i * L, L)] = jnp.where(
                v < OFF, two, two - (VOCAB2 - 1))

    def wait_gathers(b):
        pltpu.make_async_copy(
            w_hbm.at[pl.ds(0, CHUNK)], rows_v[b], sem_g[b]).wait()

    def wait_store(b):
        pltpu.make_async_copy(
            rows_v[b], out_hbm.at[pl.ds(0, CHUNK)], sem_s[b]).wait()

    def start_idx(b, t):
        pltpu.async_copy(
            idx_hbm.at[pl.ds(ibase + t * CHUNK, CHUNK)], idx_v[b], sem_i[b])

    def start_gathers(b):
        for j in range(K):
            pltpu.async_copy(
                w_hbm.at[idx_v[b].at[pl.ds(j * SUB, SUB)]],
                rows_v[b].at[pl.ds(j * SUB, SUB)], sem_g[b])

    def start_store(b, t):
        pltpu.async_copy(
            rows_v[b], out_hbm.at[pl.ds(base + t * CHUNK, CHUNK)], sem_s[b])

    for b in range(NBUF):
        start_idx(b, b)
    for b in range(NBUF):
        wait_idx(b)
        remap_idx(b)
        start_gathers(b)

    @pl.loop(0, nchunk)
    def _chunk(t):
        b = lax.rem(t, NBUF)

        def per_buf(bb):
            @pl.when(b == bb)
            def _():
                wait_gathers(bb)
                start_store(bb, t)

                @pl.when(t + NBUF < nchunk)
                def _():
                    start_idx(bb, t + NBUF)
                    wait_store(bb)
                    wait_idx(bb)
                    remap_idx(bb)
                    start_gathers(bb)

        for bb in range(NBUF):
            per_buf(bb)

    for b in range(NBUF):
        wait_store(b)

  return _gather_kernel


_OUT5 = (100, 2, 8, 32, 8, 128)


def _outperm_body(g_ref, o_ref):
    g3 = g_ref[...].reshape(128, 100, 128)
    t1 = jnp.transpose(g3, (1, 0, 2))
    t2 = jnp.transpose(t1, (0, 2, 1))
    o_ref[...] = t2.reshape(100, 2, 8, 1, 8, 128)


def _outperm(g128):
    return pl.pallas_call(
        _outperm_body,
        grid=(32,),
        in_specs=[pl.BlockSpec((12800, 128), lambda i: (i, 0))],
        out_specs=pl.BlockSpec(
            (100, 2, 8, 1, 8, 128), lambda i: (0, 0, 0, i, 0, 0)),
        out_shape=jax.ShapeDtypeStruct(_OUT5, jnp.float32),
    )(g128)


_gather = _make_gather(0, B)


def kernel(idx, weight):
    flat = idx.reshape(-1).astype(jnp.int32)
    w2 = _relayout(weight.T).reshape(VOCAB2, D)
    out = _gather(flat, w2)
    out5 = _outperm(out.reshape(B // 2, 2 * D))
    return out5.transpose(3, 5, 0, 1, 2, 4).reshape(idx.shape + (D,))
